# baseline (speedup 1.0000x reference)
import functools

import jax
import jax.numpy as jnp
from jax import lax
from jax.experimental import pallas as pl
from jax.experimental.pallas import tpu as pltpu
from jax.experimental.pallas import tpu_sc as plsc

_NBUF = 8
_SPLIT = 128


@functools.lru_cache(maxsize=None)
def _make_sc_kernel(batch: int, seq_len: int, d: int):
    info = plsc.get_sparse_core_info()
    nc, ns = info.num_cores, info.num_subcores
    nw = nc * ns
    seq_per_w = batch // nw
    rem = seq_len - _SPLIT

    mesh = plsc.VectorSubcoreMesh(core_axis_name="c", subcore_axis_name="s")

    @functools.partial(
        pl.kernel,
        mesh=mesh,
        compiler_params=pltpu.CompilerParams(use_tc_tiling_on_sc=False),
        out_type=jax.ShapeDtypeStruct((batch, seq_len, d), jnp.float32),
        scratch_types=[
            pltpu.VMEM((seq_per_w, seq_len), jnp.int32),
            [pltpu.VMEM((seq_len, d), jnp.float32)] * _NBUF,
            [pltpu.SemaphoreType.DMA] * _NBUF,
            [pltpu.SemaphoreType.DMA] * _NBUF,
            [pltpu.SemaphoreType.DMA] * _NBUF,
        ],
    )
    def k(x_hbm, tok_hbm, pos_hbm, out_hbm, idx_v, bufs, psems, gsems, ssems):
        wid = lax.axis_index("s") * nc + lax.axis_index("c")
        seq0 = wid * seq_per_w
        pltpu.sync_copy(x_hbm.at[pl.ds(seq0, seq_per_w)], idx_v)

        def prefill_start(slot):
            pltpu.async_copy(pos_hbm, bufs[slot], psems[slot])

        def prefill_wait(slot):
            pltpu.make_async_copy(pos_hbm, bufs[slot], psems[slot]).wait()

        def gather_start(g, slot):
            pltpu.async_copy(
                tok_hbm.at[idx_v.at[g, pl.ds(0, _SPLIT)]],
                bufs[slot].at[pl.ds(0, _SPLIT)],
                gsems[slot],
                add=True,
            )
            pltpu.async_copy(
                tok_hbm.at[idx_v.at[g, pl.ds(_SPLIT, rem)]],
                bufs[slot].at[pl.ds(_SPLIT, rem)],
                gsems[slot],
                add=True,
            )

        def gather_wait(g, slot):
            pltpu.make_async_copy(
                tok_hbm.at[idx_v.at[g, pl.ds(0, _SPLIT)]],
                bufs[slot].at[pl.ds(0, _SPLIT)],
                gsems[slot],
            ).wait()
            pltpu.make_async_copy(
                tok_hbm.at[idx_v.at[g, pl.ds(_SPLIT, rem)]],
                bufs[slot].at[pl.ds(_SPLIT, rem)],
                gsems[slot],
            ).wait()

        def store_start(g, slot):
            pltpu.async_copy(bufs[slot], out_hbm.at[seq0 + g], ssems[slot])

        def store_wait(g, slot):
            pltpu.make_async_copy(
                bufs[slot], out_hbm.at[seq0 + g], ssems[slot]
            ).wait()

        for s in range(3):
            prefill_start(s)
        prefill_wait(0)
        gather_start(0, 0)

        def outer(oi, carry):
            for b in range(_NBUF):
                g = oi * _NBUF + b
                s3 = (b + 3) % _NBUF
                s1 = (b + 1) % _NBUF

                @pl.when(g + 3 < seq_per_w)
                def _():
                    @pl.when(g - 5 >= 0)
                    def _():
                        store_wait(g - 5, s3)

                    prefill_start(s3)

                @pl.when(g + 1 < seq_per_w)
                def _():
                    prefill_wait(s1)
                    gather_start(g + 1, s1)

                gather_wait(g, b)
                store_start(g, b)
            return carry

        lax.fori_loop(0, seq_per_w // _NBUF, outer, 0)

        for b in range(_NBUF):
            store_wait(seq_per_w - _NBUF + b, b)

    return k


def kernel(x, token_table, pos_table):
    b, s = x.shape
    d = token_table.shape[1]
    return _make_sc_kernel(b, s, d)(
        x.astype(jnp.int32), token_table, pos_table
    )


# speedup vs baseline: 1.0053x; 1.0053x over previous
import functools

import jax
import jax.numpy as jnp
from jax import lax
from jax.experimental import pallas as pl
from jax.experimental.pallas import tpu as pltpu
from jax.experimental.pallas import tpu_sc as plsc

_NBUF = 8
_SPLIT = 128


@functools.lru_cache(maxsize=None)
def _make_sc_kernel(batch: int, seq_len: int, d: int):
    info = plsc.get_sparse_core_info()
    nc, ns = info.num_cores, info.num_subcores
    nw = nc * ns
    seq_per_w = batch // nw
    rem = seq_len - _SPLIT

    mesh = plsc.VectorSubcoreMesh(core_axis_name="c", subcore_axis_name="s")

    @functools.partial(
        pl.kernel,
        mesh=mesh,
        compiler_params=pltpu.CompilerParams(use_tc_tiling_on_sc=False),
        out_type=jax.ShapeDtypeStruct((batch, seq_len, d), jnp.float32),
        scratch_types=[
            pltpu.VMEM((seq_per_w, seq_len), jnp.int32),
            [pltpu.VMEM((seq_len, d), jnp.float32)] * _NBUF,
            [pltpu.SemaphoreType.DMA] * _NBUF,
            [pltpu.SemaphoreType.DMA] * _NBUF,
            [pltpu.SemaphoreType.DMA] * _NBUF,
        ],
    )
    def k(x_hbm, tok_hbm, pos_hbm, out_hbm, idx_v, bufs, psems, gsems, ssems):
        wid = lax.axis_index("s") * nc + lax.axis_index("c")
        seq0 = wid * seq_per_w
        pltpu.sync_copy(x_hbm.at[pl.ds(seq0, seq_per_w)], idx_v)

        def prefill_start(slot):
            pltpu.async_copy(pos_hbm, bufs[slot], psems[slot])

        def prefill_wait(slot):
            pltpu.make_async_copy(pos_hbm, bufs[slot], psems[slot]).wait()

        def gather_start(g, slot):
            pltpu.async_copy(
                tok_hbm.at[idx_v.at[g, pl.ds(0, _SPLIT)]],
                bufs[slot].at[pl.ds(0, _SPLIT)],
                gsems[slot],
                add=True,
            )
            pltpu.async_copy(
                tok_hbm.at[idx_v.at[g, pl.ds(_SPLIT, rem)]],
                bufs[slot].at[pl.ds(_SPLIT, rem)],
                gsems[slot],
                add=True,
            )

        def gather_wait(g, slot):
            pltpu.make_async_copy(
                tok_hbm.at[idx_v.at[g, pl.ds(0, _SPLIT)]],
                bufs[slot].at[pl.ds(0, _SPLIT)],
                gsems[slot],
            ).wait()
            pltpu.make_async_copy(
                tok_hbm.at[idx_v.at[g, pl.ds(_SPLIT, rem)]],
                bufs[slot].at[pl.ds(_SPLIT, rem)],
                gsems[slot],
            ).wait()

        def store_start(g, slot):
            pltpu.async_copy(bufs[slot], out_hbm.at[seq0 + g], ssems[slot])

        def store_wait(g, slot):
            pltpu.make_async_copy(
                bufs[slot], out_hbm.at[seq0 + g], ssems[slot]
            ).wait()

        for s in range(5):
            prefill_start(s)
        for s in range(3):
            prefill_wait(s)
            gather_start(s, s)

        def outer(oi, carry):
            for b in range(_NBUF):
                g = oi * _NBUF + b
                s5 = (b + 5) % _NBUF
                s3 = (b + 3) % _NBUF

                @pl.when(g + 5 < seq_per_w)
                def _():
                    @pl.when(g - 3 >= 0)
                    def _():
                        store_wait(g - 3, s5)

                    prefill_start(s5)

                @pl.when(g + 3 < seq_per_w)
                def _():
                    prefill_wait(s3)
                    gather_start(g + 3, s3)

                gather_wait(g, b)
                store_start(g, b)
            return carry

        lax.fori_loop(0, seq_per_w // _NBUF, outer, 0)

        for b in range(_NBUF):
            store_wait(seq_per_w - _NBUF + b, b)

    return k


def kernel(x, token_table, pos_table):
    b, s = x.shape
    d = token_table.shape[1]
    return _make_sc_kernel(b, s, d)(
        x.astype(jnp.int32), token_table, pos_table
    )
